# baseline (speedup 1.0000x reference)
import jax
import jax.numpy as jnp
from jax.experimental import pallas as pl

_TILE = 512


def _moe_body(x_ref, gates_ref, bg_ref, wall_ref, be_ref, p_ref, s_ref, out_ref):
    EO = p_ref.shape[1]
    E = p_ref.shape[0]
    xb = x_ref[...].astype(jnp.bfloat16)
    y_all = jnp.dot(xb, wall_ref[...], preferred_element_type=jnp.float32)
    y = y_all[:, :EO]
    g = (y_all[:, EO : EO + E] + bg_ref[...]) * gates_ref[...]
    ge = jnp.dot(g.astype(jnp.bfloat16), p_ref[...],
                 preferred_element_type=jnp.float32)
    z = (ge * y).astype(jnp.bfloat16)
    out = jnp.dot(z, s_ref[...], preferred_element_type=jnp.float32)
    out_ref[...] = out + jnp.dot(g, be_ref[...], preferred_element_type=jnp.float32)


def kernel(x, gates, W_gate, b_gate, W_experts, b_experts):
    B, D = x.shape
    E = gates.shape[1]
    O = W_experts.shape[2]
    w_flat = jnp.transpose(W_experts, (1, 0, 2)).reshape(D, E * O)
    w_all = jnp.concatenate([w_flat, W_gate], axis=1).astype(jnp.bfloat16)
    bg2 = b_gate.reshape(1, E)
    p_mat = jnp.repeat(jnp.eye(E, dtype=jnp.bfloat16), O, axis=1)
    s_mat = jnp.tile(jnp.eye(O, dtype=jnp.bfloat16), (E, 1))
    tile = _TILE if B % _TILE == 0 else B
    grid = (B // tile,)
    return pl.pallas_call(
        _moe_body,
        grid=grid,
        in_specs=[
            pl.BlockSpec((tile, D), lambda i: (i, 0)),
            pl.BlockSpec((tile, E), lambda i: (i, 0)),
            pl.BlockSpec((1, E), lambda i: (0, 0)),
            pl.BlockSpec((D, E * O + E), lambda i: (0, 0)),
            pl.BlockSpec((E, O), lambda i: (0, 0)),
            pl.BlockSpec((E, E * O), lambda i: (0, 0)),
            pl.BlockSpec((E * O, O), lambda i: (0, 0)),
        ],
        out_specs=pl.BlockSpec((tile, O), lambda i: (i, 0)),
        out_shape=jax.ShapeDtypeStruct((B, O), jnp.float32),
    )(x, gates, bg2, w_all, b_experts, p_mat, s_mat)


# speedup vs baseline: 1.1467x; 1.1467x over previous
import jax
import jax.numpy as jnp
from jax.experimental import pallas as pl

_TILE = 2048


def _moe_body(x_ref, gates_ref, bg_ref, wall_ref, be_ref, p_ref, s_ref, out_ref):
    EO = p_ref.shape[1]
    E = p_ref.shape[0]
    xb = x_ref[...].astype(jnp.bfloat16)
    y_all = jnp.dot(xb, wall_ref[...], preferred_element_type=jnp.float32)
    y = y_all[:, :EO]
    g = (y_all[:, EO : EO + E] + bg_ref[...]) * gates_ref[...]
    ge = jnp.dot(g.astype(jnp.bfloat16), p_ref[...],
                 preferred_element_type=jnp.float32)
    z = (ge * y).astype(jnp.bfloat16)
    out = jnp.dot(z, s_ref[...], preferred_element_type=jnp.float32)
    out_ref[...] = out + jnp.dot(g, be_ref[...], preferred_element_type=jnp.float32)


def kernel(x, gates, W_gate, b_gate, W_experts, b_experts):
    B, D = x.shape
    E = gates.shape[1]
    O = W_experts.shape[2]
    w_flat = jnp.transpose(W_experts, (1, 0, 2)).reshape(D, E * O)
    w_all = jnp.concatenate([w_flat, W_gate], axis=1).astype(jnp.bfloat16)
    bg2 = b_gate.reshape(1, E)
    p_mat = jnp.repeat(jnp.eye(E, dtype=jnp.bfloat16), O, axis=1)
    s_mat = jnp.tile(jnp.eye(O, dtype=jnp.bfloat16), (E, 1))
    tile = _TILE if B % _TILE == 0 else B
    grid = (B // tile,)
    return pl.pallas_call(
        _moe_body,
        grid=grid,
        in_specs=[
            pl.BlockSpec((tile, D), lambda i: (i, 0)),
            pl.BlockSpec((tile, E), lambda i: (i, 0)),
            pl.BlockSpec((1, E), lambda i: (0, 0)),
            pl.BlockSpec((D, E * O + E), lambda i: (0, 0)),
            pl.BlockSpec((E, O), lambda i: (0, 0)),
            pl.BlockSpec((E, E * O), lambda i: (0, 0)),
            pl.BlockSpec((E * O, O), lambda i: (0, 0)),
        ],
        out_specs=pl.BlockSpec((tile, O), lambda i: (i, 0)),
        out_shape=jax.ShapeDtypeStruct((B, O), jnp.float32),
    )(x, gates, bg2, w_all, b_experts, p_mat, s_mat)
